# baseline (speedup 1.0000x reference)
import jax
import jax.numpy as jnp
import numpy as np
from jax.experimental import pallas as pl
from jax.experimental.pallas import tpu as pltpu

_NUM_LEVELS = 8
_TABLE_SIZE = 2 ** 19
_FEAT = 4
_MAX_RES = 2 ** 12
_MIN_RES = 16
_OMEGA = 30.0
_PRIMES = np.array([1, 2654435761, 805459861], dtype=np.uint32)
_OFFSETS = np.array([[(i >> d) & 1 for d in range(3)] for i in range(8)])

_GROWTH = np.exp((np.log(_MAX_RES) - np.log(_MIN_RES)) / (_NUM_LEVELS - 1))
_SCALES = (_MIN_RES * _GROWTH ** np.arange(_NUM_LEVELS)).astype(np.float32)


def _corner_feats(points, table):
    scales = jnp.asarray(_SCALES)
    scaled = points[:, None, :] * scales[None, :, None]
    base = jnp.floor(scaled).astype(jnp.uint32)
    offs_u = jnp.asarray(_OFFSETS, dtype=jnp.uint32)
    corner = base[None, :, :, :] + offs_u[:, None, None, :]
    h = corner * jnp.asarray(_PRIMES)
    idx = (h[..., 0] ^ h[..., 1] ^ h[..., 2]) & jnp.uint32(_TABLE_SIZE - 1)
    idx = idx + (jnp.arange(_NUM_LEVELS, dtype=jnp.uint32) * _TABLE_SIZE)[None, None, :]
    flat_table = table.reshape(_NUM_LEVELS * _TABLE_SIZE, _FEAT)
    feats = flat_table[idx]
    n = points.shape[0]
    return feats.reshape(8, n, _NUM_LEVELS * _FEAT)


def _fused_kernel(pts_ref, feats_ref, scale_ref, *refs):
    w1 = refs[0:5]
    b1 = refs[5:10]
    w1d, b1d = refs[10], refs[11]
    w1r, b1r = refs[12], refs[13]
    w2a, w2b, b2f = refs[14], refs[15], refs[16]
    w2 = refs[17:19]
    b2 = refs[19:21]
    w2l, b2l = refs[21], refs[22]
    scalar_ref, density_ref = refs[23], refs[24]

    scale32 = scale_ref[...]
    fr = []
    for d in range(3):
        s = pts_ref[:, d:d + 1] * scale32
        fr.append(s - jnp.floor(s))
    enc = jnp.zeros_like(fr[0])
    for c in range(8):
        wc = 1.0
        for d in range(3):
            wc = wc * (fr[d] if (c >> d) & 1 else 1.0 - fr[d])
        enc = enc + wc * feats_ref[c]

    h = enc
    for li in range(5):
        h = jnp.sin(jnp.dot(h, w1[li][...], preferred_element_type=jnp.float32)
                    + b1[li][...])
    density_ref[...] = jnp.maximum(
        jnp.dot(h, w1d[...], preferred_element_type=jnp.float32) + b1d[...], 0.0)

    xr = jnp.dot(h, w1r[...], preferred_element_type=jnp.float32) + b1r[...]
    g = jnp.sin(jnp.dot(xr, w2a[...], preferred_element_type=jnp.float32)
                + jnp.dot(enc, w2b[...], preferred_element_type=jnp.float32)
                + b2f[...])
    for li in range(2):
        g = jnp.sin(jnp.dot(g, w2[li][...], preferred_element_type=jnp.float32)
                    + b2[li][...])
    scalar_ref[...] = jnp.dot(g, w2l[...], preferred_element_type=jnp.float32) + b2l[...]


def _fused_encode_mlps(points, feats, params1, params2):
    n = points.shape[0]
    B = 1024
    g2 = (n // B) // 2

    def fold(w, b, s):
        return w * s, (b * s).reshape(1, -1)

    scale32 = jnp.asarray(np.repeat(_SCALES, _FEAT).reshape(1, 32))
    ws_sin1, bs_sin1 = [], []
    for li in range(5):
        w, b = fold(params1['ws'][li], params1['bs'][li], _OMEGA)
        ws_sin1.append(w)
        bs_sin1.append(b)
    args = [scale32] + ws_sin1 + bs_sin1
    w1l, b1l = params1['ws'][5], params1['bs'][5]
    args += [w1l[:, :1], b1l[:1].reshape(1, -1), w1l[:, 1:], b1l[1:].reshape(1, -1)]
    w2f, b2f = fold(params2['ws'][0], params2['bs'][0], _OMEGA)
    args += [w2f[:15], w2f[15:], b2f]
    ws_sin2, bs_sin2 = [], []
    for li in (1, 2):
        w, b = fold(params2['ws'][li], params2['bs'][li], _OMEGA)
        ws_sin2.append(w)
        bs_sin2.append(b)
    args += ws_sin2 + bs_sin2
    args += [params2['ws'][3], params2['bs'][3].reshape(1, -1)]

    def whole(a):
        return pl.BlockSpec(a.shape, lambda i, j: (0,) * a.ndim)

    in_specs = [pl.BlockSpec((B, 3), lambda i, j: (i * g2 + j, 0)),
                pl.BlockSpec((8, B, 32), lambda i, j: (0, i * g2 + j, 0))]
    in_specs += [whole(a) for a in args]
    out_specs = [pl.BlockSpec((B, 1), lambda i, j: (i * g2 + j, 0)),
                 pl.BlockSpec((B, 1), lambda i, j: (i * g2 + j, 0))]
    out_shape = [jax.ShapeDtypeStruct((n, 1), jnp.float32),
                 jax.ShapeDtypeStruct((n, 1), jnp.float32)]

    scalar, density = pl.pallas_call(
        _fused_kernel,
        grid=(2, g2),
        in_specs=in_specs,
        out_specs=out_specs,
        out_shape=out_shape,
        compiler_params=pltpu.CompilerParams(
            dimension_semantics=("parallel", "arbitrary"),
            vmem_limit_bytes=100 * 1024 * 1024,
        ),
    )(points, feats, *args)
    return scalar, jnp.squeeze(density, -1)


def kernel(input_points, table, params1, params2):
    feats = _corner_feats(input_points, table)
    return _fused_encode_mlps(input_points, feats, params1, params2)


# speedup vs baseline: 1.0198x; 1.0198x over previous
import jax
import jax.numpy as jnp
import numpy as np
from jax.experimental import pallas as pl
from jax.experimental.pallas import tpu as pltpu

_NUM_LEVELS = 8
_TABLE_SIZE = 2 ** 19
_FEAT = 4
_MAX_RES = 2 ** 12
_MIN_RES = 16
_OMEGA = 30.0
_PRIMES = np.array([1, 2654435761, 805459861], dtype=np.uint32)
_OFFSETS = np.array([[(i >> d) & 1 for d in range(3)] for i in range(8)])

_GROWTH = np.exp((np.log(_MAX_RES) - np.log(_MIN_RES)) / (_NUM_LEVELS - 1))
_SCALES = (_MIN_RES * _GROWTH ** np.arange(_NUM_LEVELS)).astype(np.float32)


def _corner_feats(points, table):
    scales = jnp.asarray(_SCALES)
    scaled = points[None, :, :] * scales[:, None, None]
    base = jnp.floor(scaled).astype(jnp.uint32)
    offs_u = jnp.asarray(_OFFSETS, dtype=jnp.uint32)
    corner = base[:, :, None, :] + offs_u[None, None, :, :]
    h = corner * jnp.asarray(_PRIMES)
    idx = (h[..., 0] ^ h[..., 1] ^ h[..., 2]) & jnp.uint32(_TABLE_SIZE - 1)
    feats = jax.vmap(lambda t, i: t[i])(table, idx)
    n = points.shape[0]
    return feats.reshape(_NUM_LEVELS, n, 8 * _FEAT)


def _fused_kernel(pts_ref, feats_ref, bits_ref, sel_ref, *refs):
    w1 = refs[0:5]
    b1 = refs[5:10]
    w1d, b1d = refs[10], refs[11]
    w1r, b1r = refs[12], refs[13]
    w2a, w2b, b2f = refs[14], refs[15], refs[16]
    w2 = refs[17:19]
    b2 = refs[19:21]
    w2l, b2l = refs[21], refs[22]
    scalar_ref, density_ref = refs[23], refs[24]

    bits = [bits_ref[d:d + 1] for d in range(3)]
    enc = None
    for l in range(_NUM_LEVELS):
        w = None
        for d in range(3):
            s = pts_ref[:, d:d + 1] * float(_SCALES[l])
            f = s - jnp.floor(s)
            term = jnp.where(bits[d] == 1.0, f, 1.0 - f)
            w = term if w is None else w * term
        p = w * feats_ref[l]
        e = jnp.dot(p, sel_ref[l], preferred_element_type=jnp.float32)
        enc = e if enc is None else enc + e

    h = enc
    for li in range(5):
        h = jnp.sin(jnp.dot(h, w1[li][...], preferred_element_type=jnp.float32)
                    + b1[li][...])
    density_ref[...] = jnp.maximum(
        jnp.dot(h, w1d[...], preferred_element_type=jnp.float32) + b1d[...], 0.0)

    xr = jnp.dot(h, w1r[...], preferred_element_type=jnp.float32) + b1r[...]
    g = jnp.sin(jnp.dot(xr, w2a[...], preferred_element_type=jnp.float32)
                + jnp.dot(enc, w2b[...], preferred_element_type=jnp.float32)
                + b2f[...])
    for li in range(2):
        g = jnp.sin(jnp.dot(g, w2[li][...], preferred_element_type=jnp.float32)
                    + b2[li][...])
    scalar_ref[...] = jnp.dot(g, w2l[...], preferred_element_type=jnp.float32) + b2l[...]


def _fused_encode_mlps(points, feats, params1, params2):
    n = points.shape[0]
    B = 1024
    g2 = (n // B) // 2

    bits = np.zeros((3, 32), np.float32)
    for c in range(8):
        for d in range(3):
            bits[d, c * 4:(c + 1) * 4] = (c >> d) & 1
    sel = np.zeros((_NUM_LEVELS, 32, 32), np.float32)
    for l in range(_NUM_LEVELS):
        for c in range(8):
            for f in range(_FEAT):
                sel[l, c * 4 + f, l * 4 + f] = 1.0
    bits = jnp.asarray(bits)
    sel = jnp.asarray(sel)

    def fold(w, b, s):
        return w * s, (b * s).reshape(1, -1)

    ws_sin1, bs_sin1 = [], []
    for li in range(5):
        w, b = fold(params1['ws'][li], params1['bs'][li], _OMEGA)
        ws_sin1.append(w)
        bs_sin1.append(b)
    args = [bits, sel] + ws_sin1 + bs_sin1
    w1l, b1l = params1['ws'][5], params1['bs'][5]
    args += [w1l[:, :1], b1l[:1].reshape(1, -1), w1l[:, 1:], b1l[1:].reshape(1, -1)]
    w2f, b2f = fold(params2['ws'][0], params2['bs'][0], _OMEGA)
    args += [w2f[:15], w2f[15:], b2f]
    for li in (1, 2):
        w, b = fold(params2['ws'][li], params2['bs'][li], _OMEGA)
        args.append(w)
    for li in (1, 2):
        _, b = fold(params2['ws'][li], params2['bs'][li], _OMEGA)
        args.append(b)
    args += [params2['ws'][3], params2['bs'][3].reshape(1, -1)]

    def whole(a):
        return pl.BlockSpec(a.shape, lambda i, j: (0,) * a.ndim)

    in_specs = [pl.BlockSpec((B, 3), lambda i, j: (i * g2 + j, 0)),
                pl.BlockSpec((_NUM_LEVELS, B, 32), lambda i, j: (0, i * g2 + j, 0))]
    in_specs += [whole(a) for a in args]
    out_specs = [pl.BlockSpec((B, 1), lambda i, j: (i * g2 + j, 0)),
                 pl.BlockSpec((B, 1), lambda i, j: (i * g2 + j, 0))]
    out_shape = [jax.ShapeDtypeStruct((n, 1), jnp.float32),
                 jax.ShapeDtypeStruct((n, 1), jnp.float32)]

    scalar, density = pl.pallas_call(
        _fused_kernel,
        grid=(2, g2),
        in_specs=in_specs,
        out_specs=out_specs,
        out_shape=out_shape,
        compiler_params=pltpu.CompilerParams(
            dimension_semantics=("parallel", "arbitrary"),
            vmem_limit_bytes=100 * 1024 * 1024,
        ),
    )(points, feats, *args)
    return scalar, jnp.squeeze(density, -1)


def kernel(input_points, table, params1, params2):
    feats = _corner_feats(input_points, table)
    return _fused_encode_mlps(input_points, feats, params1, params2)


# speedup vs baseline: 1.7376x; 1.7038x over previous
import jax
import jax.numpy as jnp
import numpy as np
from jax.experimental import pallas as pl
from jax.experimental.pallas import tpu as pltpu

_NUM_LEVELS = 8
_TABLE_SIZE = 2 ** 19
_FEAT = 4
_MAX_RES = 2 ** 12
_MIN_RES = 16
_OMEGA = 30.0
_PRIMES = np.array([1, 2654435761, 805459861], dtype=np.uint32)
_OFFSETS = np.array([[(i >> d) & 1 for d in range(3)] for i in range(8)])

_GROWTH = np.exp((np.log(_MAX_RES) - np.log(_MIN_RES)) / (_NUM_LEVELS - 1))
_SCALES = (_MIN_RES * _GROWTH ** np.arange(_NUM_LEVELS)).astype(np.float32)


def _corner_feats(points, table):
    scales = jnp.asarray(_SCALES)
    scaled = points[None, :, :] * scales[:, None, None]
    base = jnp.floor(scaled).astype(jnp.uint32)
    offs_u = jnp.asarray(_OFFSETS, dtype=jnp.uint32)
    corner = base[:, :, None, :] + offs_u[None, None, :, :]
    h = corner * jnp.asarray(_PRIMES)
    idx = (h[..., 0] ^ h[..., 1] ^ h[..., 2]) & jnp.uint32(_TABLE_SIZE - 1)
    feats = jax.vmap(lambda t, i: t[i])(table, idx)
    frac = scaled - jnp.floor(scaled)
    offs_f = jnp.asarray(_OFFSETS, dtype=points.dtype)
    w = jnp.prod(jnp.where(offs_f[None, None] == 1,
                           frac[:, :, None, :], 1.0 - frac[:, :, None, :]),
                 axis=-1)
    weighted = w[..., None] * feats
    n = points.shape[0]
    return weighted.reshape(_NUM_LEVELS, n, 8 * _FEAT)


def _fused_kernel(feats_ref, sel_ref, *refs):
    w1 = refs[0:5]
    b1 = refs[5:10]
    w1d, b1d = refs[10], refs[11]
    w1r, b1r = refs[12], refs[13]
    w2a, w2b, b2f = refs[14], refs[15], refs[16]
    w2 = refs[17:19]
    b2 = refs[19:21]
    w2l, b2l = refs[21], refs[22]
    scalar_ref, density_ref = refs[23], refs[24]

    enc = None
    for l in range(_NUM_LEVELS):
        e = jnp.dot(feats_ref[l], sel_ref[l], preferred_element_type=jnp.float32)
        enc = e if enc is None else enc + e

    h = enc
    for li in range(5):
        h = jnp.sin(jnp.dot(h, w1[li][...], preferred_element_type=jnp.float32)
                    + b1[li][...])
    density_ref[...] = jnp.maximum(
        jnp.dot(h, w1d[...], preferred_element_type=jnp.float32) + b1d[...], 0.0)

    xr = jnp.dot(h, w1r[...], preferred_element_type=jnp.float32) + b1r[...]
    g = jnp.sin(jnp.dot(xr, w2a[...], preferred_element_type=jnp.float32)
                + jnp.dot(enc, w2b[...], preferred_element_type=jnp.float32)
                + b2f[...])
    for li in range(2):
        g = jnp.sin(jnp.dot(g, w2[li][...], preferred_element_type=jnp.float32)
                    + b2[li][...])
    scalar_ref[...] = jnp.dot(g, w2l[...], preferred_element_type=jnp.float32) + b2l[...]


def _fused_encode_mlps(feats, params1, params2):
    n = feats.shape[1]
    B = 1024
    g2 = (n // B) // 2

    sel = np.zeros((_NUM_LEVELS, 32, 32), np.float32)
    for l in range(_NUM_LEVELS):
        for c in range(8):
            for f in range(_FEAT):
                sel[l, c * 4 + f, l * 4 + f] = 1.0
    sel = jnp.asarray(sel)

    def fold(w, b, s):
        return w * s, (b * s).reshape(1, -1)

    ws_sin1, bs_sin1 = [], []
    for li in range(5):
        w, b = fold(params1['ws'][li], params1['bs'][li], _OMEGA)
        ws_sin1.append(w)
        bs_sin1.append(b)
    args = [sel] + ws_sin1 + bs_sin1
    w1l, b1l = params1['ws'][5], params1['bs'][5]
    args += [w1l[:, :1], b1l[:1].reshape(1, -1), w1l[:, 1:], b1l[1:].reshape(1, -1)]
    w2f, b2f = fold(params2['ws'][0], params2['bs'][0], _OMEGA)
    args += [w2f[:15], w2f[15:], b2f]
    for li in (1, 2):
        w, b = fold(params2['ws'][li], params2['bs'][li], _OMEGA)
        args.append(w)
    for li in (1, 2):
        _, b = fold(params2['ws'][li], params2['bs'][li], _OMEGA)
        args.append(b)
    args += [params2['ws'][3], params2['bs'][3].reshape(1, -1)]

    def whole(a):
        return pl.BlockSpec(a.shape, lambda i, j: (0,) * a.ndim)

    in_specs = [pl.BlockSpec((_NUM_LEVELS, B, 32), lambda i, j: (0, i * g2 + j, 0))]
    in_specs += [whole(a) for a in args]
    out_specs = [pl.BlockSpec((B, 1), lambda i, j: (i * g2 + j, 0)),
                 pl.BlockSpec((B, 1), lambda i, j: (i * g2 + j, 0))]
    out_shape = [jax.ShapeDtypeStruct((n, 1), jnp.float32),
                 jax.ShapeDtypeStruct((n, 1), jnp.float32)]

    scalar, density = pl.pallas_call(
        _fused_kernel,
        grid=(2, g2),
        in_specs=in_specs,
        out_specs=out_specs,
        out_shape=out_shape,
        compiler_params=pltpu.CompilerParams(
            dimension_semantics=("parallel", "arbitrary"),
            vmem_limit_bytes=100 * 1024 * 1024,
        ),
    )(feats, *args)
    return scalar, jnp.squeeze(density, -1)


def kernel(input_points, table, params1, params2):
    feats = _corner_feats(input_points, table)
    return _fused_encode_mlps(feats, params1, params2)
